# baseline (speedup 1.0000x reference)
import jax
import jax.numpy as jnp
from jax.experimental import pallas as pl


def _bmu_kernel(x_ref, w_ref, out_ref):
    x = x_ref[...]
    wt = w_ref[...].T
    wn = jnp.sum(wt * wt, axis=0, keepdims=True)
    x = x * -2.0
    xh = x.astype(jnp.bfloat16)
    xr = x - xh.astype(jnp.float32)
    xm = xr.astype(jnp.bfloat16)
    xl = (xr - xm.astype(jnp.float32)).astype(jnp.bfloat16)
    wh = wt.astype(jnp.bfloat16)
    wr = wt - wh.astype(jnp.float32)
    wm = wr.astype(jnp.bfloat16)
    wl = (wr - wm.astype(jnp.float32)).astype(jnp.bfloat16)
    xhm = jnp.concatenate([xh, xm], axis=1)
    xhl = jnp.concatenate([xh, xl], axis=1)
    f32 = jnp.float32
    dots = (jnp.dot(xhm, jnp.concatenate([wh, wm], axis=0),
                    preferred_element_type=f32)
            + jnp.dot(xhm, jnp.concatenate([wm, wh], axis=0),
                      preferred_element_type=f32)
            + jnp.dot(xhl, jnp.concatenate([wl, wh], axis=0),
                      preferred_element_type=f32))
    scores = wn + dots
    m = jnp.min(scores, axis=1, keepdims=True)
    iota = jax.lax.broadcasted_iota(jnp.int32, scores.shape, 1)
    idx = jnp.min(jnp.where(scores == m, iota, scores.shape[1]),
                  axis=1, keepdims=True)
    row = jax.lax.shift_right_logical(idx, 4)
    col = idx & 15
    lane = jax.lax.broadcasted_iota(jnp.int32, out_ref.shape, 1)
    out_ref[...] = jnp.where(lane == 0, row, col)


def kernel(x, weights):
    batch, in_size = x.shape
    w_flat = weights.reshape(-1, in_size)
    return pl.pallas_call(
        _bmu_kernel,
        out_shape=jax.ShapeDtypeStruct((batch, 2), jnp.int32),
    )(x, w_flat)
